# baseline (speedup 1.0000x reference)
import jax
import jax.numpy as jnp
from jax.experimental import pallas as pl
from jax.experimental.pallas import tpu as pltpu

_E = 8
_K = 2
_D = 768
_C = 3
_S = 512
_F = _C * _S
_N = 2
_ND = _D // _N
_EPS = 1e-9


def _moe_body(logits_ref, mask_ref, x_ref, w_ref, b_ref,
              out_ref, gl_ref, gates_ref, acc_ref):
    e = pl.program_id(0)
    n = pl.program_id(1)
    n_b = out_ref.shape[0]
    n_l = out_ref.shape[1]

    @pl.when(jnp.logical_and(e == 0, n == 0))
    def _gating():
        lg = logits_ref[...]
        mk = mask_ref[...]
        m = jnp.where(mk == 1.0, 1.0, 0.0).astype(jnp.float32)
        z = lg - jnp.max(lg, axis=1, keepdims=True)
        ez = jnp.exp(z)
        probs = ez / jnp.sum(ez, axis=1, keepdims=True)
        pm = probs * m
        iota = jax.lax.broadcasted_iota(jnp.int32, pm.shape, 1)
        m1 = jnp.max(pm, axis=1, keepdims=True)
        a1 = jnp.min(jnp.where(pm == m1, iota, _E), axis=1, keepdims=True)
        pm2 = jnp.where(iota == a1, -1.0, pm)
        m2 = jnp.max(pm2, axis=1, keepdims=True)
        a2 = jnp.min(jnp.where(pm2 == m2, iota, _E), axis=1, keepdims=True)
        topk = jnp.logical_or(iota == a1, iota == a2)
        gts = jnp.where(topk, pm, 0.0)
        dn = jnp.sum(gts, axis=1, keepdims=True) + _EPS
        gates_ref[...] = gts / dn
        s = jnp.sum(pm) / jnp.float32(n_b)
        gl_ref[...] = ((1.0 - s) * (1.0 - s)).reshape(1, 1)

    onehot = (jax.lax.broadcasted_iota(jnp.int32, (_E, 1), 0) == e
              ).astype(jnp.float32)
    g_col = jnp.dot(gates_ref[...], onehot)

    y = jnp.dot(x_ref[...], w_ref[0].astype(jnp.bfloat16),
                preferred_element_type=jnp.float32)
    y3 = y.reshape(n_b, n_l, _ND) + b_ref[pl.ds(e, 1), :].reshape(1, 1, _ND)
    contrib = (g_col.reshape(n_b, 1, 1) * y3).reshape(1, n_b, n_l, _ND)

    @pl.when(e == 0)
    def _init():
        acc_ref[pl.ds(n, 1)] = contrib

    @pl.when(e > 0)
    def _acc():
        acc_ref[pl.ds(n, 1)] += contrib

    @pl.when(jnp.logical_and(e == _E - 1, n == _N - 1))
    def _fin():
        full = jnp.concatenate([acc_ref[0], acc_ref[1]], axis=-1)
        out_ref[...] = full.astype(jnp.bfloat16)


def kernel(cycle_curve_data, logits, moe_masks, W, b):
    B, L = cycle_curve_data.shape[0], cycle_curve_data.shape[1]
    x = cycle_curve_data.astype(jnp.bfloat16).reshape(B * L, _F)

    out, gl = pl.pallas_call(
        _moe_body,
        grid=(_E, _N),
        in_specs=[
            pl.BlockSpec((B, _E), lambda e, n: (0, 0)),
            pl.BlockSpec((B, _E), lambda e, n: (0, 0)),
            pl.BlockSpec((B * L, _F), lambda e, n: (0, 0)),
            pl.BlockSpec((1, _F, _ND), lambda e, n: (e, 0, n)),
            pl.BlockSpec((_E, _ND), lambda e, n: (0, n)),
        ],
        out_specs=[
            pl.BlockSpec((B, L, _D), lambda e, n: (0, 0, 0)),
            pl.BlockSpec((1, 1), lambda e, n: (0, 0)),
        ],
        out_shape=[
            jax.ShapeDtypeStruct((B, L, _D), jnp.bfloat16),
            jax.ShapeDtypeStruct((1, 1), jnp.float32),
        ],
        scratch_shapes=[
            pltpu.VMEM((B, _E), jnp.float32),
            pltpu.VMEM((_N, B, L, _ND), jnp.float32),
        ],
        compiler_params=pltpu.CompilerParams(
            dimension_semantics=("arbitrary", "arbitrary"),
        ),
    )(logits, moe_masks, x, W, b)

    return out, gl[0, 0]


# speedup vs baseline: 1.1572x; 1.1572x over previous
import jax
import jax.numpy as jnp
from jax.experimental import pallas as pl
from jax.experimental.pallas import tpu as pltpu

_E = 8
_K = 2
_D = 768
_C = 3
_S = 512
_F = _C * _S
_EPS = 1e-9


def _moe_body(logits_ref, mask_ref, x_ref, w_ref, b_ref,
              out_ref, gl_ref, gates_ref, acc_ref):
    e = pl.program_id(0)
    n_b = out_ref.shape[0]
    n_l = out_ref.shape[1]

    @pl.when(e == 0)
    def _prologue():
        lg = logits_ref[...]
        mk = mask_ref[...]
        m = jnp.where(mk == 1.0, 1.0, 0.0).astype(jnp.float32)
        z = lg - jnp.max(lg, axis=1, keepdims=True)
        ez = jnp.exp(z)
        probs = ez / jnp.sum(ez, axis=1, keepdims=True)
        pm = probs * m
        iota = jax.lax.broadcasted_iota(jnp.int32, pm.shape, 1)
        m1 = jnp.max(pm, axis=1, keepdims=True)
        a1 = jnp.min(jnp.where(pm == m1, iota, _E), axis=1, keepdims=True)
        pm2 = jnp.where(iota == a1, -1.0, pm)
        m2 = jnp.max(pm2, axis=1, keepdims=True)
        a2 = jnp.min(jnp.where(pm2 == m2, iota, _E), axis=1, keepdims=True)
        topk = jnp.logical_or(iota == a1, iota == a2)
        gts = jnp.where(topk, pm, 0.0)
        dn = jnp.sum(gts, axis=1, keepdims=True) + _EPS
        gates_ref[...] = gts / dn
        s = jnp.sum(pm) / jnp.float32(n_b)
        gl_ref[...] = ((1.0 - s) * (1.0 - s)).reshape(1, 1)

    onehot = (jax.lax.broadcasted_iota(jnp.int32, (_E, 1), 0) == e
              ).astype(jnp.float32)
    g_col = jnp.dot(gates_ref[...], onehot)

    y = jnp.dot(x_ref[...], w_ref[0].astype(jnp.bfloat16),
                preferred_element_type=jnp.float32)
    y3 = y.reshape(n_b, n_l, _D) + b_ref[pl.ds(e, 1), :].reshape(1, 1, _D)
    contrib = g_col.reshape(n_b, 1, 1) * y3

    @pl.when(e == 0)
    def _init():
        acc_ref[...] = contrib

    @pl.when(e > 0)
    def _acc():
        acc_ref[...] += contrib

    @pl.when(e == _E - 1)
    def _fin():
        out_ref[...] = acc_ref[...].astype(jnp.bfloat16)


def kernel(cycle_curve_data, logits, moe_masks, W, b):
    B, L = cycle_curve_data.shape[0], cycle_curve_data.shape[1]
    x = cycle_curve_data.astype(jnp.bfloat16).reshape(B * L, _F)

    out, gl = pl.pallas_call(
        _moe_body,
        grid=(_E,),
        in_specs=[
            pl.BlockSpec((B, _E), lambda e: (0, 0)),
            pl.BlockSpec((B, _E), lambda e: (0, 0)),
            pl.BlockSpec((B * L, _F), lambda e: (0, 0)),
            pl.BlockSpec((1, _F, _D), lambda e: (e, 0, 0)),
            pl.BlockSpec((_E, _D), lambda e: (0, 0)),
        ],
        out_specs=[
            pl.BlockSpec((B, L, _D), lambda e: (0, 0, 0)),
            pl.BlockSpec((1, 1), lambda e: (0, 0)),
        ],
        out_shape=[
            jax.ShapeDtypeStruct((B, L, _D), jnp.bfloat16),
            jax.ShapeDtypeStruct((1, 1), jnp.float32),
        ],
        scratch_shapes=[
            pltpu.VMEM((B, _E), jnp.float32),
            pltpu.VMEM((B, L, _D), jnp.float32),
        ],
        compiler_params=pltpu.CompilerParams(
            dimension_semantics=("arbitrary",),
        ),
    )(logits, moe_masks, x, W, b)

    return out, gl[0, 0]
